# Initial kernel scaffold; baseline (speedup 1.0000x reference)
import jax
import jax.numpy as jnp
from jax.experimental import pallas as pl


def kernel(x, edge_index, W, b):
    raise NotImplementedError("write your pallas kernel here")


# speedup vs baseline: 14.0167x; 14.0167x over previous
import functools

import jax
import jax.numpy as jnp
from jax import lax
from jax.experimental import pallas as pl
from jax.experimental.pallas import tpu as pltpu
from jax.experimental.pallas import tpu_sc as plsc

NC = 2
NS = 16
L = 16
K = 128


def _node_pad(n):
    return -(-(n + 1) // (NS * K)) * (NS * K)


def _deg_call(dst_pad, n):
    e_pad = dst_pad.shape[0]
    nw = NC * NS
    epw = e_pad // nw
    n_pad = _node_pad(n)
    mesh = plsc.VectorSubcoreMesh(core_axis_name="c", subcore_axis_name="s")

    @functools.partial(
        pl.kernel,
        mesh=mesh,
        out_type=jax.ShapeDtypeStruct((nw, n_pad), jnp.float32),
        scratch_types=[
            pltpu.VMEM((epw,), jnp.int32),
            pltpu.VMEM((n_pad,), jnp.float32),
        ],
        compiler_params=pltpu.CompilerParams(needs_layout_passes=False),
    )
    def deg_kernel(dst_hbm, out_hbm, idx_v, deg_v):
        c = lax.axis_index("c")
        s = lax.axis_index("s")
        wid = c * NS + s
        zeros = jnp.zeros((L,), jnp.float32)

        def zbody(i, carry):
            deg_v[pl.ds(i * L, L)] = zeros
            return carry

        lax.fori_loop(0, n_pad // L, zbody, 0)
        pltpu.sync_copy(dst_hbm.at[pl.ds(wid * epw, epw)], idx_v)
        ones = jnp.ones((L,), jnp.float32)

        def hbody(i, carry):
            idx = idx_v[pl.ds(i * L, L)]
            plsc.addupdate_scatter(deg_v, [idx], ones)
            return carry

        lax.fori_loop(0, epw // L, hbody, 0)
        pltpu.sync_copy(deg_v, out_hbm.at[wid])

    return deg_kernel(dst_pad)


def _scatter_call(hp, src_pad, dst_pad):
    n, d = hp.shape
    e_pad = src_pad.shape[0]
    nw = NC * NS
    epw = e_pad // nw
    nchunks = epw // K
    n_pad = _node_pad(n)
    zpt = n_pad // NS
    mesh = plsc.VectorSubcoreMesh(core_axis_name="c", subcore_axis_name="s")

    @functools.partial(
        pl.kernel,
        mesh=mesh,
        out_type=jax.ShapeDtypeStruct((NC, n_pad, d), jnp.float32),
        scratch_types=[
            pltpu.VMEM((K,), jnp.int32),
            pltpu.VMEM((K,), jnp.int32),
            pltpu.VMEM((K, d), jnp.float32),
            pltpu.VMEM_SHARED((n_pad, d), jnp.float32),
            pltpu.SemaphoreType.DMA,
        ],
        compiler_params=pltpu.CompilerParams(needs_layout_passes=False),
    )
    def scat_kernel(hp_hbm, src_hbm, dst_hbm, out_hbm,
                    sidx_v, didx_v, rows_v, acc_sh, sem):
        c = lax.axis_index("c")
        s = lax.axis_index("s")
        wid = c * NS + s
        zeros = jnp.zeros((L,), jnp.float32)
        cols = d // L

        def zbody(i, carry):
            r = i // cols
            col = (i % cols) * L
            rows_v[r, pl.ds(col, L)] = zeros
            return carry

        lax.fori_loop(0, K * cols, zbody, 0)
        for k2 in range(zpt // K):
            pltpu.sync_copy(rows_v, acc_sh.at[pl.ds(s * zpt + k2 * K, K)])
        plsc.subcore_barrier()

        def body(j, carry):
            eb = wid * epw + j * K
            pltpu.sync_copy(src_hbm.at[pl.ds(eb, K)], sidx_v)
            pltpu.sync_copy(dst_hbm.at[pl.ds(eb, K)], didx_v)
            pltpu.async_copy(hp_hbm.at[sidx_v], rows_v, sem).wait()
            pltpu.sync_copy(rows_v, acc_sh.at[didx_v], add=True)
            return carry

        lax.fori_loop(0, nchunks, body, 0)
        plsc.subcore_barrier()
        pltpu.sync_copy(acc_sh.at[pl.ds(s * zpt, zpt)],
                        out_hbm.at[c, pl.ds(s * zpt, zpt)])

    return scat_kernel(hp, src_pad, dst_pad)


def _dinv_cols(dp_block):
    nw = dp_block.shape[1]
    ones = jnp.ones((nw, 1), jnp.float32)
    deg = jnp.dot(dp_block, ones, preferred_element_type=jnp.float32)
    return lax.rsqrt(deg + 1.0)


def _matmul_call(x, W, deg_t):
    n, d = x.shape
    nw = deg_t.shape[1]
    blk = 2000
    grid = n // blk

    def body(x_ref, w_ref, dp_ref, o_ref):
        dinv = _dinv_cols(dp_ref[...])
        h = jnp.dot(x_ref[...], w_ref[...],
                    preferred_element_type=jnp.float32,
                    precision=lax.Precision.HIGHEST)
        o_ref[...] = h * dinv

    return pl.pallas_call(
        body,
        grid=(grid,),
        in_specs=[
            pl.BlockSpec((blk, d), lambda i: (i, 0)),
            pl.BlockSpec((d, d), lambda i: (0, 0)),
            pl.BlockSpec((blk, nw), lambda i: (i, 0)),
        ],
        out_specs=pl.BlockSpec((blk, d), lambda i: (i, 0)),
        out_shape=jax.ShapeDtypeStruct((n, d), jnp.float32),
    )(x, W, deg_t)


def _epilogue_call(x, hp, s_parts, deg_t, b2):
    n, d = x.shape
    nw = deg_t.shape[1]
    blk = 2000
    grid = n // blk

    def body(x_ref, hp_ref, s0_ref, s1_ref, dp_ref, b_ref, o_ref):
        dinv = _dinv_cols(dp_ref[...])
        stot = (s0_ref[...].reshape(blk, d) + s1_ref[...].reshape(blk, d)
                + hp_ref[...])
        agg = stot * dinv + b_ref[...]
        o_ref[...] = x_ref[...] + jnp.maximum(agg, 0.0)

    return pl.pallas_call(
        body,
        grid=(grid,),
        in_specs=[
            pl.BlockSpec((blk, d), lambda i: (i, 0)),
            pl.BlockSpec((blk, d), lambda i: (i, 0)),
            pl.BlockSpec((1, blk, d), lambda i: (0, i, 0)),
            pl.BlockSpec((1, blk, d), lambda i: (1, i, 0)),
            pl.BlockSpec((blk, nw), lambda i: (i, 0)),
            pl.BlockSpec((1, d), lambda i: (0, 0)),
        ],
        out_specs=pl.BlockSpec((blk, d), lambda i: (i, 0)),
        out_shape=jax.ShapeDtypeStruct((n, d), jnp.float32),
    )(x, hp, s_parts, s_parts, deg_t, b2)


def kernel(x, edge_index, W, b):
    n, d = x.shape
    e = edge_index.shape[1]
    cpt = NC * NS * K
    e_pad = -(-e // cpt) * cpt
    src = edge_index[0]
    dst = edge_index[1]
    pad = e_pad - e
    if pad:
        src = jnp.concatenate([src, jnp.zeros((pad,), jnp.int32)])
        dst = jnp.concatenate([dst, jnp.full((pad,), n, jnp.int32)])

    deg_parts = _deg_call(dst, n)
    deg_t = deg_parts.T
    hp = _matmul_call(x, W, deg_t)
    s_parts = _scatter_call(hp, src, dst)
    y = _epilogue_call(x, hp, s_parts, deg_t, b.reshape(1, d))
    return y
